# baseline (speedup 1.0000x reference)
import jax
import jax.numpy as jnp
from jax.experimental import pallas as pl
from jax.experimental.pallas import tpu as pltpu


def _mlp_step(x_hbm, gup_hbm, dp_hbm, o_hbm,
              xbuf, obuf, gubuf, wgu_stage, wdp_stage, wgu16, wdp16,
              lsem, ssem, wgsem, wdsem):
    ei = pl.program_id(0)
    j = pl.program_id(1)
    ne = pl.num_programs(0)
    nsteps = 2 * ne
    k = 2 * ei + j
    slot = jax.lax.rem(k, 2)
    nslot = jax.lax.rem(k + 1, 2)
    eslot = jax.lax.rem(ei, 2)
    h2 = xbuf.shape[3]

    def xload(ke, kj, sl):
        return pltpu.make_async_copy(
            x_hbm.at[:, :, ke, pl.ds(kj * h2, h2)],
            xbuf.at[sl], lsem.at[sl])

    def wguload(ke, kj, sl):
        return pltpu.make_async_copy(
            gup_hbm.at[ke, pl.ds(kj * h2, h2), :],
            wgu_stage.at[sl], wgsem.at[sl])

    def wdpload(ke, sl):
        return pltpu.make_async_copy(
            dp_hbm.at[ke], wdp_stage.at[sl], wdsem.at[sl])

    @pl.when(k == 0)
    def _():
        xload(0, 0, 0).start()
        wguload(0, 0, 0).start()
        wdpload(0, 0).start()

    @pl.when(k + 1 < nsteps)
    def _():
        kn = k + 1
        xload(kn // 2, jax.lax.rem(kn, 2), nslot).start()
        wguload(kn // 2, jax.lax.rem(kn, 2), nslot).start()

    @pl.when((j == 0) & (ei + 1 < ne))
    def _():
        wdpload(ei + 1, jax.lax.rem(ei + 1, 2)).start()

    wguload(ei, j, slot).wait()
    wgu16[...] = wgu_stage[slot].astype(jnp.bfloat16)
    xload(ei, j, slot).wait()

    bb, gg = xbuf.shape[1], xbuf.shape[2]
    rows = bb * gg
    ih = wdp16.shape[0]
    hh = o_hbm.shape[3]
    xe = xbuf[slot].reshape(rows, h2).astype(jnp.bfloat16)
    part = jnp.dot(xe, wgu16[...], preferred_element_type=jnp.float32)

    @pl.when(j == 0)
    def _():
        gubuf[...] = part

    @pl.when(j == 1)
    def _():
        wdpload(ei, eslot).wait()
        wdp16[...] = wdp_stage[eslot].astype(jnp.bfloat16)
        gu = gubuf[...] + part
        inter = (jax.nn.silu(gu[:, :ih]) * gu[:, ih:]).astype(jnp.bfloat16)

        def store(sl):
            return pltpu.make_async_copy(
                obuf.at[sl], o_hbm.at[:, :, ei, :], ssem.at[sl])

        @pl.when(ei >= 2)
        def _():
            store(eslot).wait()

        obuf[eslot] = jnp.dot(inter, wdp16[...],
                              preferred_element_type=jnp.float32
                              ).reshape(bb, gg, hh)
        store(eslot).start()

        @pl.when(ei == ne - 1)
        def _():
            store(eslot).wait()

            @pl.when(ne >= 2)
            def _():
                store(jax.lax.rem(ei + 1, 2)).wait()


def kernel(x, gate_up_proj, down_proj):
    b, n, h = x.shape
    e, _, i2 = gate_up_proj.shape
    i = i2 // 2
    g = n // e
    h2 = h // 2
    x4 = x.reshape(b, g, e, h)
    out4 = pl.pallas_call(
        _mlp_step,
        grid=(e, 2),
        in_specs=[
            pl.BlockSpec(memory_space=pl.ANY),
            pl.BlockSpec(memory_space=pl.ANY),
            pl.BlockSpec(memory_space=pl.ANY),
        ],
        out_specs=pl.BlockSpec(memory_space=pl.ANY),
        out_shape=jax.ShapeDtypeStruct((b, g, e, h), jnp.float32),
        scratch_shapes=[
            pltpu.VMEM((2, b, g, h2), jnp.float32),
            pltpu.VMEM((2, b, g, h), jnp.float32),
            pltpu.VMEM((b * g, i2), jnp.float32),
            pltpu.VMEM((2, h2, i2), jnp.float32),
            pltpu.VMEM((2, i, h), jnp.float32),
            pltpu.VMEM((h2, i2), jnp.bfloat16),
            pltpu.VMEM((i, h), jnp.bfloat16),
            pltpu.SemaphoreType.DMA((2,)),
            pltpu.SemaphoreType.DMA((2,)),
            pltpu.SemaphoreType.DMA((2,)),
            pltpu.SemaphoreType.DMA((2,)),
        ],
    )(x4, gate_up_proj, down_proj)
    return out4.reshape(b, n, h)


# speedup vs baseline: 1.3082x; 1.3082x over previous
import jax
import jax.numpy as jnp
from jax.experimental import pallas as pl
from jax.experimental.pallas import tpu as pltpu


def _mlp_step(x_hbm, gup_hbm, dp_hbm, o_hbm,
              xbuf, obuf, wgu_stage, wdp_stage, wgu16, wdp16,
              lsem, ssem, wgsem, wdsem):
    ei = pl.program_id(0)
    ne = pl.num_programs(0)
    slot = jax.lax.rem(ei, 2)
    nslot = jax.lax.rem(ei + 1, 2)

    def xload(kk, sl):
        return pltpu.make_async_copy(
            x_hbm.at[:, :, kk, :], xbuf.at[sl], lsem.at[sl])

    def wguload(kk, sl):
        return pltpu.make_async_copy(
            gup_hbm.at[kk], wgu_stage.at[sl], wgsem.at[sl])

    def wdpload(kk, sl):
        return pltpu.make_async_copy(
            dp_hbm.at[kk], wdp_stage.at[sl], wdsem.at[sl])

    @pl.when(ei == 0)
    def _():
        xload(ei, slot).start()
        wguload(ei, slot).start()
        wdpload(ei, slot).start()

    @pl.when(ei + 1 < ne)
    def _():
        xload(ei + 1, nslot).start()
        wguload(ei + 1, nslot).start()
        wdpload(ei + 1, nslot).start()

    wguload(ei, slot).wait()
    wdpload(ei, slot).wait()
    wgu16[...] = wgu_stage[slot].astype(jnp.bfloat16)
    wdp16[...] = wdp_stage[slot].astype(jnp.bfloat16)

    xload(ei, slot).wait()

    bb, gg, hh = xbuf.shape[1], xbuf.shape[2], xbuf.shape[3]
    ih = wdp16.shape[0]
    xe = xbuf[slot].reshape(bb * gg, hh).astype(jnp.bfloat16)
    gu = jnp.dot(xe, wgu16[...], preferred_element_type=jnp.float32)
    inter = (jax.nn.silu(gu[:, :ih]) * gu[:, ih:]).astype(jnp.bfloat16)

    def store(sl):
        return pltpu.make_async_copy(
            obuf.at[sl], o_hbm.at[:, :, ei, :], ssem.at[sl])

    @pl.when(ei >= 2)
    def _():
        store(slot).wait()

    obuf[slot] = jnp.dot(inter, wdp16[...],
                         preferred_element_type=jnp.float32).reshape(bb, gg, hh)
    store(slot).start()

    @pl.when(ei == ne - 1)
    def _():
        store(slot).wait()
        store(nslot).wait()


def kernel(x, gate_up_proj, down_proj):
    b, n, h = x.shape
    e, _, i2 = gate_up_proj.shape
    i = i2 // 2
    g = n // e
    x4 = x.reshape(b, g, e, h)
    out4 = pl.pallas_call(
        _mlp_step,
        grid=(e,),
        in_specs=[
            pl.BlockSpec(memory_space=pl.ANY),
            pl.BlockSpec(memory_space=pl.ANY),
            pl.BlockSpec(memory_space=pl.ANY),
        ],
        out_specs=pl.BlockSpec(memory_space=pl.ANY),
        out_shape=jax.ShapeDtypeStruct((b, g, e, h), jnp.float32),
        scratch_shapes=[
            pltpu.VMEM((2, b, g, h), jnp.float32),
            pltpu.VMEM((2, b, g, h), jnp.float32),
            pltpu.VMEM((2, h, i2), jnp.float32),
            pltpu.VMEM((2, i, h), jnp.float32),
            pltpu.VMEM((h, i2), jnp.bfloat16),
            pltpu.VMEM((i, h), jnp.bfloat16),
            pltpu.SemaphoreType.DMA((2,)),
            pltpu.SemaphoreType.DMA((2,)),
            pltpu.SemaphoreType.DMA((2,)),
            pltpu.SemaphoreType.DMA((2,)),
        ],
    )(x4, gate_up_proj, down_proj)
    return out4.reshape(b, n, h)
